# baseline (speedup 1.0000x reference)
import jax
import jax.numpy as jnp
from jax import lax
from jax.experimental import pallas as pl
from jax.experimental.pallas import tpu as pltpu
from jax.experimental.pallas import tpu_sc as plsc

N_POINTS = 100000
N_ELEMENTS = 200000
ELEMENT_SIZE = 3
DIM = 2

NC, NS, LANES = 2, 16, 16
NW = NC * NS
E_TILE = N_ELEMENTS // NW
CHUNK_E = 128
CHUNK_R = CHUNK_E * ELEMENT_SIZE
N_CHUNKS = -(-E_TILE // CHUNK_E)
E_TILE_PAD = N_CHUNKS * CHUNK_E
ACC_N = 100352
OUT_SLICE = ACC_N // NS

W1_OFF = 0
B1_OFF = 48
W2_OFF = 56
B2_OFF = 120
W3_OFF = 128
B3_OFF = 192
W4_OFF = 200
B4_OFF = 224
N_W = 227
W_LEN = 240


def _sig(x):
    return 1.0 / (1.0 + jnp.exp(-x))


def _sc_body(pts_hbm, el_hbm, wbuf_hbm, out_hbm,
             el_v, gat_v, vals_v, wv, wb, zbuf, ha, hb, acc_sh, sem_g, sem_s):
    c = lax.axis_index("c")
    s = lax.axis_index("s")
    wid = c * NS + s
    LAST_E = N_ELEMENTS - (NW - 1) * E_TILE_PAD

    for i in range(OUT_SLICE // LANES):
        zbuf[pl.ds(i * LANES, LANES)] = jnp.zeros((LANES,), jnp.float32)
    pltpu.sync_copy(zbuf, acc_sh.at[pl.ds(s * OUT_SLICE, OUT_SLICE)])

    @pl.when(wid < NW - 1)
    def _():
        for v in range(ELEMENT_SIZE):
            pltpu.sync_copy(
                el_hbm.at[pl.ds(v * N_ELEMENTS + wid * E_TILE_PAD,
                                E_TILE_PAD)],
                el_v.at[v])

    @pl.when(wid == NW - 1)
    def _():
        for v in range(ELEMENT_SIZE):
            pltpu.sync_copy(
                el_hbm.at[pl.ds(v * N_ELEMENTS + (NW - 1) * E_TILE_PAD,
                                LAST_E)],
                el_v.at[v].at[pl.ds(0, LAST_E)])
            for i in range((E_TILE_PAD - LAST_E) // LANES):
                el_v.at[v][pl.ds(LAST_E + i * LANES, LANES)] = (
                    jnp.zeros((LANES,), jnp.int32))

    pltpu.sync_copy(wbuf_hbm, wv)
    iota = lax.iota(jnp.int32, LANES)

    def yidx_step(mi, carry):
        for u in range(8):
            sl = pl.ds((mi * 8 + u) * LANES, LANES)
            for v in range(ELEMENT_SIZE):
                el_v.at[ELEMENT_SIZE + v][sl] = el_v.at[v][sl] + N_POINTS
        return carry

    lax.fori_loop(0, E_TILE_PAD // (LANES * 8), yidx_step, 0)
    plsc.subcore_barrier()

    for b in range(0, W_LEN, LANES):
        vec = wv[pl.ds(b, LANES)]
        for j in range(LANES):
            if b + j < N_W:
                wb.at[b + j][pl.ds(0, LANES)] = (
                    jnp.zeros((LANES,), jnp.float32) + vec[j])

    def wrow(k):
        return wb.at[k][pl.ds(0, LANES)]

    col0 = jnp.zeros((LANES,), jnp.int32)
    col1 = col0 + 1
    n_real = jnp.minimum(E_TILE_PAD, N_ELEMENTS - wid * E_TILE_PAD)

    def fire_gather(ci, par):
        for q in range(2 * ELEMENT_SIZE):
            pltpu.async_copy(
                pts_hbm.at[el_v.at[q].at[pl.ds(ci * CHUNK_E, CHUNK_E)]],
                gat_v.at[par].at[q], sem_g)

    def drain_gather(par):
        for q in range(2 * ELEMENT_SIZE):
            pltpu.make_async_copy(
                pts_hbm.at[el_v.at[q].at[pl.ds(0, CHUNK_E)]],
                gat_v.at[par].at[q], sem_g).wait()

    def fire_scatter(ci, par):
        for v in range(ELEMENT_SIZE):
            pltpu.async_copy(
                vals_v.at[par].at[v],
                acc_sh.at[el_v.at[v].at[pl.ds(ci * CHUNK_E, CHUNK_E)]],
                sem_s, add=True)

    def drain_scatter(par):
        for v in range(ELEMENT_SIZE):
            pltpu.make_async_copy(
                vals_v.at[par].at[v],
                acc_sh.at[el_v.at[v].at[pl.ds(0, CHUNK_E)]], sem_s).wait()

    fire_gather(0, 0)

    def chunk(ci, carry):
        par = lax.rem(ci, 2)
        drain_gather(par)

        @pl.when(ci < N_CHUNKS - 1)
        def _():
            fire_gather(ci + 1, 1 - par)

        @pl.when(ci >= 2)
        def _():
            drain_scatter(par)

        gref = gat_v.at[par]
        vref = vals_v.at[par]
        NG = CHUNK_E // LANES
        for g in range(NG):
            sl = pl.ds(g * LANES, LANES)
            ins = [gref.at[v + c2 * ELEMENT_SIZE][sl]
                   for v in range(ELEMENT_SIZE) for c2 in range(2)]
            for j in range(8):
                ha.at[j][sl] = _sig(
                    sum(ins[i] * wrow(W1_OFF + i * 8 + j) for i in range(6))
                    + wrow(B1_OFF + j))
        for g in range(NG):
            sl = pl.ds(g * LANES, LANES)
            hi = [ha.at[i][sl] for i in range(8)]
            for j in range(8):
                hb.at[j][sl] = _sig(
                    sum(hi[i] * wrow(W2_OFF + i * 8 + j) for i in range(8))
                    + wrow(B2_OFF + j))
        for g in range(NG):
            sl = pl.ds(g * LANES, LANES)
            hi = [hb.at[i][sl] for i in range(8)]
            for j in range(8):
                ha.at[j][sl] = _sig(
                    sum(hi[i] * wrow(W3_OFF + i * 8 + j) for i in range(8))
                    + wrow(B3_OFF + j))
        for g in range(NG):
            sl = pl.ds(g * LANES, LANES)
            rows = g * LANES + iota
            hi = [ha.at[i][sl] for i in range(8)]
            live = (ci * CHUNK_E + g * LANES + iota) < n_real
            for v in range(ELEMENT_SIZE):
                o = _sig(sum(hi[i] * wrow(W4_OFF + i * 3 + v) for i in range(8))
                         + wrow(B4_OFF + v))
                o = jnp.where(live, o, 0.0)
                plsc.store_scatter(vref.at[v], [rows], o)

        fire_scatter(ci, par)
        return carry

    lax.fori_loop(0, N_CHUNKS, chunk, 0)
    drain_scatter(1)
    drain_scatter(0)
    plsc.subcore_barrier()

    pltpu.sync_copy(acc_sh.at[pl.ds(s * OUT_SLICE, OUT_SLICE)],
                    out_hbm.at[c, pl.ds(s * OUT_SLICE, OUT_SLICE)])


@jax.jit
def _sc_call(pts, el, wbuf):
    mesh = plsc.VectorSubcoreMesh(core_axis_name="c", subcore_axis_name="s")
    return pl.kernel(
        _sc_body,
        out_type=jax.ShapeDtypeStruct((NC, ACC_N), jnp.float32),
        mesh=mesh,
        scratch_types=[
            pltpu.VMEM((2 * ELEMENT_SIZE, E_TILE_PAD), jnp.int32),
            pltpu.VMEM((2, 2 * ELEMENT_SIZE, CHUNK_E), jnp.float32),
            pltpu.VMEM((2, ELEMENT_SIZE, CHUNK_E), jnp.float32),
            pltpu.VMEM((W_LEN,), jnp.float32),
            pltpu.VMEM((W_LEN, LANES), jnp.float32),
            pltpu.VMEM((OUT_SLICE,), jnp.float32),
            pltpu.VMEM((8, CHUNK_E), jnp.float32),
            pltpu.VMEM((8, CHUNK_E), jnp.float32),
            pltpu.VMEM_SHARED((ACC_N,), jnp.float32),
            pltpu.SemaphoreType.DMA,
            pltpu.SemaphoreType.DMA,
        ],
        compiler_params=pltpu.CompilerParams(
            needs_layout_passes=False, use_tc_tiling_on_sc=False),
    )(pts, el, wbuf)


def _combine_body(p_hbm, o_hbm, a_v, b_v, sem):
    c = lax.axis_index("c")
    s = lax.axis_index("s")
    wid = c * NS + s
    W = ACC_N // NW
    cp_a = pltpu.async_copy(p_hbm.at[0, pl.ds(wid * W, W)], a_v, sem)
    cp_b = pltpu.async_copy(p_hbm.at[1, pl.ds(wid * W, W)], b_v, sem)
    cp_a.wait()
    cp_b.wait()
    for i in range(W // LANES):
        sl = pl.ds(i * LANES, LANES)
        a_v[sl] = a_v[sl] + b_v[sl]
    pltpu.sync_copy(a_v, o_hbm.at[pl.ds(wid * W, W)])


@jax.jit
def _combine(partials):
    mesh = plsc.VectorSubcoreMesh(core_axis_name="c", subcore_axis_name="s")
    out = pl.kernel(
        _combine_body,
        out_type=jax.ShapeDtypeStruct((ACC_N,), jnp.float32),
        mesh=mesh,
        scratch_types=[
            pltpu.VMEM((ACC_N // NW,), jnp.float32),
            pltpu.VMEM((ACC_N // NW,), jnp.float32),
            pltpu.SemaphoreType.DMA,
        ],
        compiler_params=pltpu.CompilerParams(
            needs_layout_passes=False, use_tc_tiling_on_sc=False),
    )(partials)
    return out[:N_POINTS]


def kernel(points, elements, W1, b1, W2, b2, W3, b3, W4, b4):
    wbuf = jnp.concatenate([
        W1.reshape(-1), b1, W2.reshape(-1), b2, W3.reshape(-1), b3,
        W4.reshape(-1), b4, jnp.zeros((W_LEN - N_W,), jnp.float32)])
    el_flat = jnp.swapaxes(elements, 0, 1).reshape(-1)
    pts_flat = jnp.swapaxes(points, 0, 1).reshape(-1)
    partials = _sc_call(pts_flat, el_flat, wbuf)
    return _combine(partials)


# speedup vs baseline: 1.1243x; 1.1243x over previous
import jax
import jax.numpy as jnp
from jax import lax
from jax.experimental import pallas as pl
from jax.experimental.pallas import tpu as pltpu
from jax.experimental.pallas import tpu_sc as plsc

N_POINTS = 100000
N_ELEMENTS = 200000
ELEMENT_SIZE = 3
DIM = 2

NC, NS, LANES = 2, 16, 16
NW = NC * NS
E_TILE = N_ELEMENTS // NW
CHUNK_E = 128
CHUNK_R = CHUNK_E * ELEMENT_SIZE
N_CHUNKS = -(-E_TILE // CHUNK_E)
E_TILE_PAD = N_CHUNKS * CHUNK_E
ACC_N = 100352
OUT_SLICE = ACC_N // NS

W1_OFF = 0
B1_OFF = 48
W2_OFF = 56
B2_OFF = 120
W3_OFF = 128
B3_OFF = 192
W4_OFF = 200
B4_OFF = 224
N_W = 227
W_LEN = 240


def _sig(x):
    return 1.0 / (1.0 + jnp.exp(-x))


def _sc_body(pts_hbm, el_hbm, wbuf_hbm, out_hbm,
             el_v, gat_v, vals_v, wv, wb, zbuf, ha, hb, acc_sh, sem_g, sem_s):
    c = lax.axis_index("c")
    s = lax.axis_index("s")
    wid = c * NS + s
    LAST_E = N_ELEMENTS - (NW - 1) * E_TILE_PAD

    for i in range(OUT_SLICE // LANES):
        zbuf[pl.ds(i * LANES, LANES)] = jnp.zeros((LANES,), jnp.float32)
    pltpu.sync_copy(zbuf, acc_sh.at[pl.ds(s * OUT_SLICE, OUT_SLICE)])

    @pl.when(wid < NW - 1)
    def _():
        for v in range(ELEMENT_SIZE):
            pltpu.sync_copy(
                el_hbm.at[pl.ds(v * N_ELEMENTS + wid * E_TILE_PAD,
                                E_TILE_PAD)],
                el_v.at[v])

    @pl.when(wid == NW - 1)
    def _():
        for v in range(ELEMENT_SIZE):
            pltpu.sync_copy(
                el_hbm.at[pl.ds(v * N_ELEMENTS + (NW - 1) * E_TILE_PAD,
                                LAST_E)],
                el_v.at[v].at[pl.ds(0, LAST_E)])
            for i in range((E_TILE_PAD - LAST_E) // LANES):
                el_v.at[v][pl.ds(LAST_E + i * LANES, LANES)] = (
                    jnp.zeros((LANES,), jnp.int32))

    pltpu.sync_copy(wbuf_hbm, wv)
    iota = lax.iota(jnp.int32, LANES)

    def yidx_step(mi, carry):
        for u in range(8):
            sl = pl.ds((mi * 8 + u) * LANES, LANES)
            for v in range(ELEMENT_SIZE):
                el_v.at[ELEMENT_SIZE + v][sl] = el_v.at[v][sl] + N_POINTS
        return carry

    lax.fori_loop(0, E_TILE_PAD // (LANES * 8), yidx_step, 0)
    plsc.subcore_barrier()

    for b in range(0, W_LEN, LANES):
        vec = wv[pl.ds(b, LANES)]
        for j in range(LANES):
            if b + j < N_W:
                wb.at[b + j][pl.ds(0, LANES)] = (
                    jnp.zeros((LANES,), jnp.float32) + vec[j])

    def wrow(k):
        return wb.at[k][pl.ds(0, LANES)]

    col0 = jnp.zeros((LANES,), jnp.int32)
    col1 = col0 + 1
    n_real = jnp.minimum(E_TILE_PAD, N_ELEMENTS - wid * E_TILE_PAD)

    def fire_gather(ci, par):
        for q in range(2 * ELEMENT_SIZE):
            pltpu.async_copy(
                pts_hbm.at[el_v.at[q].at[pl.ds(ci * CHUNK_E, CHUNK_E)]],
                gat_v.at[par].at[q], sem_g)

    def drain_gather(par):
        for q in range(2 * ELEMENT_SIZE):
            pltpu.make_async_copy(
                pts_hbm.at[el_v.at[q].at[pl.ds(0, CHUNK_E)]],
                gat_v.at[par].at[q], sem_g).wait()

    def fire_scatter(ci, par):
        for v in range(ELEMENT_SIZE):
            pltpu.async_copy(
                vals_v.at[par].at[v],
                acc_sh.at[el_v.at[v].at[pl.ds(ci * CHUNK_E, CHUNK_E)]],
                sem_s, add=True)

    def drain_scatter(par):
        for v in range(ELEMENT_SIZE):
            pltpu.make_async_copy(
                vals_v.at[par].at[v],
                acc_sh.at[el_v.at[v].at[pl.ds(0, CHUNK_E)]], sem_s).wait()

    fire_gather(0, 0)

    def chunk(ci, carry):
        par = lax.rem(ci, 2)
        drain_gather(par)

        @pl.when(ci < N_CHUNKS - 1)
        def _():
            fire_gather(ci + 1, 1 - par)

        @pl.when(ci >= 2)
        def _():
            drain_scatter(par)

        gref = gat_v.at[par]
        vref = vals_v.at[par]
        NG = CHUNK_E // LANES
        w1 = [[wrow(W1_OFF + i * 8 + j) for i in range(6)] for j in range(8)]
        c1 = [wrow(B1_OFF + j) for j in range(8)]
        for g in range(NG):
            sl = pl.ds(g * LANES, LANES)
            ins = [gref.at[v + c2 * ELEMENT_SIZE][sl]
                   for v in range(ELEMENT_SIZE) for c2 in range(2)]
            for j in range(8):
                ha.at[j][sl] = _sig(
                    sum(ins[i] * w1[j][i] for i in range(6)) + c1[j])
        w2 = [[wrow(W2_OFF + i * 8 + j) for i in range(8)] for j in range(8)]
        c2b = [wrow(B2_OFF + j) for j in range(8)]
        for g in range(NG):
            sl = pl.ds(g * LANES, LANES)
            hi = [ha.at[i][sl] for i in range(8)]
            for j in range(8):
                hb.at[j][sl] = _sig(
                    sum(hi[i] * w2[j][i] for i in range(8)) + c2b[j])
        w3 = [[wrow(W3_OFF + i * 8 + j) for i in range(8)] for j in range(8)]
        c3 = [wrow(B3_OFF + j) for j in range(8)]
        for g in range(NG):
            sl = pl.ds(g * LANES, LANES)
            hi = [hb.at[i][sl] for i in range(8)]
            for j in range(8):
                ha.at[j][sl] = _sig(
                    sum(hi[i] * w3[j][i] for i in range(8)) + c3[j])
        w4 = [[wrow(W4_OFF + i * 3 + v) for i in range(8)]
              for v in range(ELEMENT_SIZE)]
        c4 = [wrow(B4_OFF + v) for v in range(ELEMENT_SIZE)]
        for g in range(NG):
            sl = pl.ds(g * LANES, LANES)
            rows = g * LANES + iota
            hi = [ha.at[i][sl] for i in range(8)]
            live = (ci * CHUNK_E + g * LANES + iota) < n_real
            for v in range(ELEMENT_SIZE):
                o = _sig(sum(hi[i] * w4[v][i] for i in range(8)) + c4[v])
                o = jnp.where(live, o, 0.0)
                plsc.store_scatter(vref.at[v], [rows], o)

        fire_scatter(ci, par)
        return carry

    lax.fori_loop(0, N_CHUNKS, chunk, 0)
    drain_scatter(1)
    drain_scatter(0)
    plsc.subcore_barrier()

    pltpu.sync_copy(acc_sh.at[pl.ds(s * OUT_SLICE, OUT_SLICE)],
                    out_hbm.at[c, pl.ds(s * OUT_SLICE, OUT_SLICE)])


@jax.jit
def _sc_call(pts, el, wbuf):
    mesh = plsc.VectorSubcoreMesh(core_axis_name="c", subcore_axis_name="s")
    return pl.kernel(
        _sc_body,
        out_type=jax.ShapeDtypeStruct((NC, ACC_N), jnp.float32),
        mesh=mesh,
        scratch_types=[
            pltpu.VMEM((2 * ELEMENT_SIZE, E_TILE_PAD), jnp.int32),
            pltpu.VMEM((2, 2 * ELEMENT_SIZE, CHUNK_E), jnp.float32),
            pltpu.VMEM((2, ELEMENT_SIZE, CHUNK_E), jnp.float32),
            pltpu.VMEM((W_LEN,), jnp.float32),
            pltpu.VMEM((W_LEN, LANES), jnp.float32),
            pltpu.VMEM((OUT_SLICE,), jnp.float32),
            pltpu.VMEM((8, CHUNK_E), jnp.float32),
            pltpu.VMEM((8, CHUNK_E), jnp.float32),
            pltpu.VMEM_SHARED((ACC_N,), jnp.float32),
            pltpu.SemaphoreType.DMA,
            pltpu.SemaphoreType.DMA,
        ],
        compiler_params=pltpu.CompilerParams(
            needs_layout_passes=False, use_tc_tiling_on_sc=False),
    )(pts, el, wbuf)


def _combine_body(p_hbm, o_hbm, a_v, b_v, sem):
    c = lax.axis_index("c")
    s = lax.axis_index("s")
    wid = c * NS + s
    W = ACC_N // NW
    cp_a = pltpu.async_copy(p_hbm.at[0, pl.ds(wid * W, W)], a_v, sem)
    cp_b = pltpu.async_copy(p_hbm.at[1, pl.ds(wid * W, W)], b_v, sem)
    cp_a.wait()
    cp_b.wait()
    for i in range(W // LANES):
        sl = pl.ds(i * LANES, LANES)
        a_v[sl] = a_v[sl] + b_v[sl]
    pltpu.sync_copy(a_v, o_hbm.at[pl.ds(wid * W, W)])


@jax.jit
def _combine(partials):
    mesh = plsc.VectorSubcoreMesh(core_axis_name="c", subcore_axis_name="s")
    out = pl.kernel(
        _combine_body,
        out_type=jax.ShapeDtypeStruct((ACC_N,), jnp.float32),
        mesh=mesh,
        scratch_types=[
            pltpu.VMEM((ACC_N // NW,), jnp.float32),
            pltpu.VMEM((ACC_N // NW,), jnp.float32),
            pltpu.SemaphoreType.DMA,
        ],
        compiler_params=pltpu.CompilerParams(
            needs_layout_passes=False, use_tc_tiling_on_sc=False),
    )(partials)
    return out[:N_POINTS]


def kernel(points, elements, W1, b1, W2, b2, W3, b3, W4, b4):
    wbuf = jnp.concatenate([
        W1.reshape(-1), b1, W2.reshape(-1), b2, W3.reshape(-1), b3,
        W4.reshape(-1), b4, jnp.zeros((W_LEN - N_W,), jnp.float32)])
    el_flat = jnp.swapaxes(elements, 0, 1).reshape(-1)
    pts_flat = jnp.swapaxes(points, 0, 1).reshape(-1)
    partials = _sc_call(pts_flat, el_flat, wbuf)
    return _combine(partials)


# speedup vs baseline: 1.1297x; 1.0048x over previous
import jax
import jax.numpy as jnp
from jax import lax
from jax.experimental import pallas as pl
from jax.experimental.pallas import tpu as pltpu
from jax.experimental.pallas import tpu_sc as plsc

N_POINTS = 100000
N_ELEMENTS = 200000
ELEMENT_SIZE = 3
DIM = 2

NC, NS, LANES = 2, 16, 16
NW = NC * NS
E_TILE = N_ELEMENTS // NW
CHUNK_E = 128
CHUNK_R = CHUNK_E * ELEMENT_SIZE
N_CHUNKS = -(-E_TILE // CHUNK_E)
E_TILE_PAD = N_CHUNKS * CHUNK_E
ACC_N = 100352
OUT_SLICE = ACC_N // NS

W1_OFF = 0
B1_OFF = 48
W2_OFF = 56
B2_OFF = 120
W3_OFF = 128
B3_OFF = 192
W4_OFF = 200
B4_OFF = 224
N_W = 227
W_LEN = 240


def _sig(x):
    return 1.0 / (1.0 + jnp.exp(-x))


def _sc_body(pts_hbm, el_hbm, wbuf_hbm, out_hbm,
             el_v, gat_v, vals_v, wv, wb, zbuf, ha, hb, acc_sh, sem_g, sem_s):
    c = lax.axis_index("c")
    s = lax.axis_index("s")
    wid = c * NS + s
    LAST_E = N_ELEMENTS - (NW - 1) * E_TILE_PAD

    for i in range(OUT_SLICE // LANES):
        zbuf[pl.ds(i * LANES, LANES)] = jnp.zeros((LANES,), jnp.float32)
    pltpu.sync_copy(zbuf, acc_sh.at[pl.ds(s * OUT_SLICE, OUT_SLICE)])

    @pl.when(wid < NW - 1)
    def _():
        for v in range(ELEMENT_SIZE):
            pltpu.sync_copy(
                el_hbm.at[pl.ds(v * N_ELEMENTS + wid * E_TILE_PAD,
                                E_TILE_PAD)],
                el_v.at[v])

    @pl.when(wid == NW - 1)
    def _():
        for v in range(ELEMENT_SIZE):
            pltpu.sync_copy(
                el_hbm.at[pl.ds(v * N_ELEMENTS + (NW - 1) * E_TILE_PAD,
                                LAST_E)],
                el_v.at[v].at[pl.ds(0, LAST_E)])
            for i in range((E_TILE_PAD - LAST_E) // LANES):
                el_v.at[v][pl.ds(LAST_E + i * LANES, LANES)] = (
                    jnp.zeros((LANES,), jnp.int32))

    pltpu.sync_copy(wbuf_hbm, wv)
    iota = lax.iota(jnp.int32, LANES)

    def yidx_step(mi, carry):
        for u in range(8):
            sl = pl.ds((mi * 8 + u) * LANES, LANES)
            for v in range(ELEMENT_SIZE):
                el_v.at[ELEMENT_SIZE + v][sl] = el_v.at[v][sl] + N_POINTS
        return carry

    lax.fori_loop(0, E_TILE_PAD // (LANES * 8), yidx_step, 0)
    plsc.subcore_barrier()

    for b in range(0, W_LEN, LANES):
        vec = wv[pl.ds(b, LANES)]
        for j in range(LANES):
            if b + j < N_W:
                wb.at[b + j][pl.ds(0, LANES)] = (
                    jnp.zeros((LANES,), jnp.float32) + vec[j])

    def wrow(k):
        return wb.at[k][pl.ds(0, LANES)]

    n_real = jnp.minimum(E_TILE_PAD, N_ELEMENTS - wid * E_TILE_PAD)

    def fire_gather(ci, par):
        for q in range(2 * ELEMENT_SIZE):
            pltpu.async_copy(
                pts_hbm.at[el_v.at[q].at[pl.ds(ci * CHUNK_E, CHUNK_E)]],
                gat_v.at[par].at[q], sem_g)

    def drain_gather(par):
        for q in range(2 * ELEMENT_SIZE):
            pltpu.make_async_copy(
                pts_hbm.at[el_v.at[q].at[pl.ds(0, CHUNK_E)]],
                gat_v.at[par].at[q], sem_g).wait()

    def fire_scatter(ci, par):
        for v in range(ELEMENT_SIZE):
            pltpu.async_copy(
                vals_v.at[par].at[v],
                acc_sh.at[el_v.at[v].at[pl.ds(ci * CHUNK_E, CHUNK_E)]],
                sem_s, add=True)

    def drain_scatter(par):
        for v in range(ELEMENT_SIZE):
            pltpu.make_async_copy(
                vals_v.at[par].at[v],
                acc_sh.at[el_v.at[v].at[pl.ds(0, CHUNK_E)]], sem_s).wait()

    fire_gather(0, 0)

    def chunk(ci, carry):
        par = lax.rem(ci, 2)
        drain_gather(par)

        @pl.when(ci < N_CHUNKS - 1)
        def _():
            fire_gather(ci + 1, 1 - par)

        @pl.when(ci >= 2)
        def _():
            drain_scatter(par)

        gref = gat_v.at[par]
        vref = vals_v.at[par]
        NG = CHUNK_E // LANES
        w1 = [[wrow(W1_OFF + i * 8 + j) for i in range(6)] for j in range(8)]
        c1 = [wrow(B1_OFF + j) for j in range(8)]
        for g in range(NG):
            sl = pl.ds(g * LANES, LANES)
            ins = [gref.at[v + c2 * ELEMENT_SIZE][sl]
                   for v in range(ELEMENT_SIZE) for c2 in range(2)]
            for j in range(8):
                ha.at[j][sl] = _sig(
                    sum(ins[i] * w1[j][i] for i in range(6)) + c1[j])
        w2 = [[wrow(W2_OFF + i * 8 + j) for i in range(8)] for j in range(8)]
        c2b = [wrow(B2_OFF + j) for j in range(8)]
        for g in range(NG):
            sl = pl.ds(g * LANES, LANES)
            hi = [ha.at[i][sl] for i in range(8)]
            for j in range(8):
                hb.at[j][sl] = _sig(
                    sum(hi[i] * w2[j][i] for i in range(8)) + c2b[j])
        w3 = [[wrow(W3_OFF + i * 8 + j) for i in range(8)] for j in range(8)]
        c3 = [wrow(B3_OFF + j) for j in range(8)]
        for g in range(NG):
            sl = pl.ds(g * LANES, LANES)
            hi = [hb.at[i][sl] for i in range(8)]
            for j in range(8):
                ha.at[j][sl] = _sig(
                    sum(hi[i] * w3[j][i] for i in range(8)) + c3[j])
        w4 = [[wrow(W4_OFF + i * 3 + v) for i in range(8)]
              for v in range(ELEMENT_SIZE)]
        c4 = [wrow(B4_OFF + v) for v in range(ELEMENT_SIZE)]
        for g in range(NG):
            sl = pl.ds(g * LANES, LANES)
            rows = g * LANES + iota
            hi = [ha.at[i][sl] for i in range(8)]
            live = (ci * CHUNK_E + g * LANES + iota) < n_real
            for v in range(ELEMENT_SIZE):
                o = _sig(sum(hi[i] * w4[v][i] for i in range(8)) + c4[v])
                o = jnp.where(live, o, 0.0)
                plsc.store_scatter(vref.at[v], [rows], o)

        fire_scatter(ci, par)
        return carry

    lax.fori_loop(0, N_CHUNKS, chunk, 0)
    drain_scatter(1)
    drain_scatter(0)
    plsc.subcore_barrier()

    pltpu.sync_copy(acc_sh.at[pl.ds(s * OUT_SLICE, OUT_SLICE)],
                    out_hbm.at[c, pl.ds(s * OUT_SLICE, OUT_SLICE)])


@jax.jit
def _sc_call(pts, el, wbuf):
    mesh = plsc.VectorSubcoreMesh(core_axis_name="c", subcore_axis_name="s")
    return pl.kernel(
        _sc_body,
        out_type=jax.ShapeDtypeStruct((NC, ACC_N), jnp.float32),
        mesh=mesh,
        scratch_types=[
            pltpu.VMEM((2 * ELEMENT_SIZE, E_TILE_PAD), jnp.int32),
            pltpu.VMEM((2, 2 * ELEMENT_SIZE, CHUNK_E), jnp.float32),
            pltpu.VMEM((2, ELEMENT_SIZE, CHUNK_E), jnp.float32),
            pltpu.VMEM((W_LEN,), jnp.float32),
            pltpu.VMEM((W_LEN, LANES), jnp.float32),
            pltpu.VMEM((OUT_SLICE,), jnp.float32),
            pltpu.VMEM((8, CHUNK_E), jnp.float32),
            pltpu.VMEM((8, CHUNK_E), jnp.float32),
            pltpu.VMEM_SHARED((ACC_N,), jnp.float32),
            pltpu.SemaphoreType.DMA,
            pltpu.SemaphoreType.DMA,
        ],
        compiler_params=pltpu.CompilerParams(
            needs_layout_passes=False, use_tc_tiling_on_sc=False),
    )(pts, el, wbuf)


def _combine_body(p_hbm, o_hbm, a_v, b_v, sem):
    c = lax.axis_index("c")
    s = lax.axis_index("s")
    wid = c * NS + s
    W = ACC_N // NW
    cp_a = pltpu.async_copy(p_hbm.at[0, pl.ds(wid * W, W)], a_v, sem)
    cp_b = pltpu.async_copy(p_hbm.at[1, pl.ds(wid * W, W)], b_v, sem)
    cp_a.wait()
    cp_b.wait()
    for i in range(W // LANES):
        sl = pl.ds(i * LANES, LANES)
        a_v[sl] = a_v[sl] + b_v[sl]
    pltpu.sync_copy(a_v, o_hbm.at[pl.ds(wid * W, W)])


@jax.jit
def _combine(partials):
    mesh = plsc.VectorSubcoreMesh(core_axis_name="c", subcore_axis_name="s")
    out = pl.kernel(
        _combine_body,
        out_type=jax.ShapeDtypeStruct((ACC_N,), jnp.float32),
        mesh=mesh,
        scratch_types=[
            pltpu.VMEM((ACC_N // NW,), jnp.float32),
            pltpu.VMEM((ACC_N // NW,), jnp.float32),
            pltpu.SemaphoreType.DMA,
        ],
        compiler_params=pltpu.CompilerParams(
            needs_layout_passes=False, use_tc_tiling_on_sc=False),
    )(partials)
    return out[:N_POINTS]


def kernel(points, elements, W1, b1, W2, b2, W3, b3, W4, b4):
    wbuf = jnp.concatenate([
        W1.reshape(-1), b1, W2.reshape(-1), b2, W3.reshape(-1), b3,
        W4.reshape(-1), b4, jnp.zeros((W_LEN - N_W,), jnp.float32)])
    el_flat = jnp.swapaxes(elements, 0, 1).reshape(-1)
    pts_flat = jnp.swapaxes(points, 0, 1).reshape(-1)
    partials = _sc_call(pts_flat, el_flat, wbuf)
    return _combine(partials)
